# Initial kernel scaffold; baseline (speedup 1.0000x reference)
import jax
import jax.numpy as jnp
from jax.experimental import pallas as pl


def kernel(x, routing_weights, selected_experts, W1, W2):
    raise NotImplementedError("write your pallas kernel here")


# speedup vs baseline: 2.7958x; 2.7958x over previous
import functools

import jax
import jax.numpy as jnp
from jax import lax
from jax.experimental import pallas as pl
from jax.experimental.pallas import tpu as pltpu


def _routing_metadata(selected_experts, routing_weights, E, B, NB):
    T, K = selected_experts.shape
    S = T * K
    flat_e = selected_experts.reshape(-1)
    gates = routing_weights.reshape(-1).astype(jnp.float32)
    onehot = (flat_e[:, None] == jnp.arange(E, dtype=jnp.int32)[None, :])
    counts = jnp.sum(onehot.astype(jnp.int32), axis=0)
    pc = ((counts + B - 1) // B) * B
    ends = jnp.cumsum(pc)
    starts = ends - pc
    rank_all = jnp.cumsum(onehot.astype(jnp.int32), axis=0) - 1
    rank = jnp.take_along_axis(rank_all, flat_e[:, None], axis=1)[:, 0]
    pos = starts[flat_e] + rank
    token_ids = (jnp.arange(S, dtype=jnp.int32) // K)
    NP = NB * B
    tok_padded = jnp.zeros((NP,), jnp.int32).at[pos].set(token_ids)
    gate_padded = jnp.zeros((NP,), jnp.float32).at[pos].set(gates)
    brow = jnp.arange(NB, dtype=jnp.int32) * B
    block_expert = jnp.sum((brow[:, None] >= ends[None, :]).astype(jnp.int32),
                           axis=1)
    block_expert = jnp.minimum(block_expert, E - 1)
    nact = ends[E - 1] // B
    block_valid = (jnp.arange(NB, dtype=jnp.int32) < nact).astype(jnp.int32)
    pos_tk = pos.reshape(T, K)
    return tok_padded, gate_padded, block_expert, block_valid, pos_tk


def _gmm_body(NF, be_ref, bv_ref, xs_ref, w1h_ref, w1g_ref, w2_ref, g_ref,
              out_ref):
    f = pl.program_id(1)
    b = pl.program_id(0)

    @pl.when(f == 0)
    def _():
        out_ref[...] = jnp.zeros_like(out_ref)

    @pl.when(bv_ref[b] > 0)
    def _():
        x = xs_ref[...]
        h = lax.dot_general(x, w1h_ref[0], (((1,), (1,)), ((), ())),
                            preferred_element_type=jnp.float32)
        g = lax.dot_general(x, w1g_ref[0], (((1,), (1,)), ((), ())),
                            preferred_element_type=jnp.float32)
        hg = h * (g * jax.nn.sigmoid(g))
        yp = lax.dot_general(hg, w2_ref[0], (((1,), (1,)), ((), ())),
                             preferred_element_type=jnp.float32)
        out_ref[...] += yp

    @pl.when(f == NF - 1)
    def _():
        out_ref[...] *= g_ref[0, 0][:, None]


def _grouped_mlp(xs, w1, w2, gate_padded, block_expert, block_valid,
                 B, TF, interpret=False):
    E, F2, H = w1.shape
    F = F2 // 2
    NF = F // TF
    NP = xs.shape[0]
    NB = NP // B
    gates3 = gate_padded.reshape(NB, 1, B)

    grid_spec = pltpu.PrefetchScalarGridSpec(
        num_scalar_prefetch=2,
        grid=(NB, NF),
        in_specs=[
            pl.BlockSpec((B, H), lambda b, f, be, bv: (b, 0)),
            pl.BlockSpec((1, TF, H), lambda b, f, be, bv: (be[b], f, 0)),
            pl.BlockSpec((1, TF, H), lambda b, f, be, bv: (be[b], NF + f, 0)),
            pl.BlockSpec((1, H, TF), lambda b, f, be, bv: (be[b], 0, f)),
            pl.BlockSpec((1, 1, B), lambda b, f, be, bv: (b, 0, 0)),
        ],
        out_specs=pl.BlockSpec((B, H), lambda b, f, be, bv: (b, 0)),
    )
    return pl.pallas_call(
        functools.partial(_gmm_body, NF),
        grid_spec=grid_spec,
        out_shape=jax.ShapeDtypeStruct((NP, H), jnp.float32),
        compiler_params=pltpu.CompilerParams(
            dimension_semantics=("arbitrary", "arbitrary")),
        interpret=interpret,
    )(block_expert, block_valid, xs, w1, w1, w2, gates3)


def _fused_experts(x, routing_weights, selected_experts, W1, W2,
                   interpret=False):
    x_shape = x.shape
    H = x_shape[-1]
    xf = x.reshape(-1, H)
    T, K = selected_experts.shape
    E = W1.shape[0]
    B = 256
    S = T * K
    NB = (S + E * (B - 1) + B - 1) // B
    tok_padded, gate_padded, block_expert, block_valid, pos_tk = \
        _routing_metadata(selected_experts, routing_weights, E, B, NB)
    xs = jnp.take(xf, tok_padded, axis=0)
    ys = _grouped_mlp(xs, W1, W2, gate_padded, block_expert, block_valid,
                      B, 512, interpret=interpret)
    y = ys[pos_tk[:, 0]]
    for k in range(1, K):
        y = y + ys[pos_tk[:, k]]
    return y.reshape(*x_shape[:-1], H)


def kernel(x, routing_weights, selected_experts, W1, W2):
    return _fused_experts(x, routing_weights, selected_experts, W1, W2)
